# Initial kernel scaffold; baseline (speedup 1.0000x reference)
import jax
import jax.numpy as jnp
from jax.experimental import pallas as pl


def kernel(e_nodes, edge_index, x_nodes, W_self, W_agg, b):
    raise NotImplementedError("write your pallas kernel here")


# speedup vs baseline: 4.6479x; 4.6479x over previous
import functools

import jax
import jax.numpy as jnp
from jax import lax
from jax.experimental import pallas as pl
from jax.experimental.pallas import tpu as pltpu
from jax.experimental.pallas import tpu_sc as plsc

NC = 2
NS = 16
NW = NC * NS
K = 128


def _sc_agg_body(nch, rpt, h_hbm, src_hbm, dst_hbm, zeros_hbm, agg_hbm,
                 src_v, dst_v, rows_v, acc_sh, sem):
    c = lax.axis_index("c")
    s = lax.axis_index("s")
    wid = c * NS + s
    pltpu.sync_copy(zeros_hbm, acc_sh.at[pl.ds(s * rpt, rpt)])
    pltpu.sync_copy(src_hbm.at[wid], src_v)
    pltpu.sync_copy(dst_hbm.at[wid], dst_v)
    plsc.subcore_barrier()

    def chunk(j, _):
        pltpu.async_copy(h_hbm.at[src_v.at[j]], rows_v, sem).wait()
        pltpu.sync_copy(rows_v, acc_sh.at[dst_v.at[j]], add=True)
        return _

    lax.fori_loop(0, nch, chunk, 0)
    plsc.subcore_barrier()
    pltpu.sync_copy(acc_sh.at[pl.ds(s * rpt, rpt)], agg_hbm.at[c, pl.ds(s * rpt, rpt)])


@functools.lru_cache(maxsize=None)
def _make_sc_agg(n, d, nch):
    nacc = (n // 128 + 1) * 128
    rpt = nacc // NS
    mesh = plsc.VectorSubcoreMesh(core_axis_name="c", subcore_axis_name="s")
    return pl.kernel(
        functools.partial(_sc_agg_body, nch, rpt),
        out_type=jax.ShapeDtypeStruct((NC, nacc, d), jnp.float32),
        mesh=mesh,
        scratch_types=[
            pltpu.VMEM((nch, K), jnp.int32),
            pltpu.VMEM((nch, K), jnp.int32),
            pltpu.VMEM((K, d), jnp.float32),
            pltpu.VMEM_SHARED((nacc, d), jnp.float32),
            pltpu.SemaphoreType.DMA,
        ],
    )


def _dense_body(h_ref, a0_ref, a1_ref, ws_ref, wa_ref, b_ref, o_ref):
    agg = a0_ref[...] + a1_ref[...]
    acc = jnp.dot(h_ref[...], ws_ref[...], preferred_element_type=jnp.float32)
    acc = acc + jnp.dot(agg, wa_ref[...], preferred_element_type=jnp.float32)
    o_ref[...] = jnp.maximum(acc + b_ref[...], 0.0)


@functools.lru_cache(maxsize=None)
def _make_dense(n, d, bl):
    return pl.pallas_call(
        _dense_body,
        grid=(n // bl,),
        in_specs=[
            pl.BlockSpec((bl, d), lambda i: (i, 0)),
            pl.BlockSpec((bl, d), lambda i: (i, 0)),
            pl.BlockSpec((bl, d), lambda i: (i, 0)),
            pl.BlockSpec((d, d), lambda i: (0, 0)),
            pl.BlockSpec((d, d), lambda i: (0, 0)),
            pl.BlockSpec((1, d), lambda i: (0, 0)),
        ],
        out_specs=pl.BlockSpec((bl, d), lambda i: (i, 0)),
        out_shape=jax.ShapeDtypeStruct((n, d), jnp.float32),
    )


def kernel(e_nodes, edge_index, x_nodes, W_self, W_agg, b):
    B, n, d = e_nodes.shape
    e = edge_index.shape[1]
    num_layers = W_self.shape[0]

    ept = -(-e // NW)
    nch = -(-ept // K)
    pad = NW * nch * K - e

    src = edge_index[0]
    dst = edge_index[1]
    src_p = jnp.concatenate([src, jnp.zeros((pad,), jnp.int32)]).reshape(NW, nch, K)
    dst_p = jnp.concatenate([dst, jnp.full((pad,), n, jnp.int32)]).reshape(NW, nch, K)
    nacc = (n // 128 + 1) * 128
    zeros = jnp.zeros((nacc // NS, d), jnp.float32)

    sc_agg = _make_sc_agg(n, d, nch)
    dense = _make_dense(n, d, 1000)

    h = e_nodes[0]
    for l in range(num_layers):
        agg = sc_agg(h, src_p, dst_p, zeros)
        h = dense(h, agg[0], agg[1], W_self[l], W_agg[l], b[l].reshape(1, d))

    out = jnp.concatenate([x_nodes[..., :4], h[None]], axis=2)
    return (out, edge_index)
